# baseline (speedup 1.0000x reference)
import functools
import math

import jax
import jax.numpy as jnp
from jax import lax
from jax.experimental import pallas as pl
from jax.experimental.pallas import tpu as pltpu
from jax.experimental.pallas import tpu_sc as plsc

VOCAB = 1000000
HDIM = 64
BATCH = 4096
SEQLEN = 200

NUM_WORKERS = 32
ROWS = BATCH * SEQLEN
ROWS_PER_WORKER = ROWS // NUM_WORKERS
CHUNK = SEQLEN
NCHUNK = ROWS_PER_WORKER // CHUNK

GATHER_SPLITS = ((0, 104), (104, 96))


def _sinusoidal_pe(length, d_model):
    pos = jnp.arange(length, dtype=jnp.float32)[:, None]
    i = jnp.arange(0, d_model, 2, dtype=jnp.float32)
    div = jnp.exp(-(math.log(10000.0)) * i / d_model)
    pe = jnp.zeros((length, d_model), dtype=jnp.float32)
    pe = pe.at[:, 0::2].set(jnp.sin(pos * div))
    pe = pe.at[:, 1::2].set(jnp.cos(pos * div))
    return pe


def _make_sc_kernel():
    mesh = plsc.VectorSubcoreMesh(core_axis_name="c", subcore_axis_name="s",
                                  num_cores=2, num_subcores=16)

    @functools.partial(
        pl.kernel,
        mesh=mesh,
        out_type=jax.ShapeDtypeStruct((ROWS, HDIM), jnp.float32),
        scratch_types=[
            pltpu.VMEM((ROWS_PER_WORKER,), jnp.int32),
            pltpu.VMEM((CHUNK, HDIM), jnp.float32),
            pltpu.VMEM((CHUNK, HDIM), jnp.float32),
            pltpu.VMEM((SEQLEN, HDIM), jnp.float32),
            pltpu.SemaphoreType.DMA,
            pltpu.SemaphoreType.DMA,
        ],
        compiler_params=pltpu.CompilerParams(use_tc_tiling_on_sc=False),
    )
    def k(idx_hbm, pe_hbm, table_hbm, out_hbm,
          idx_v, buf0, buf1, pe_v, gsem0, gsem1):
        wid = lax.axis_index("s") * 2 + lax.axis_index("c")
        base = wid * ROWS_PER_WORKER
        bufs = (buf0, buf1)
        gsems = (gsem0, gsem1)

        pltpu.sync_copy(pe_hbm, pe_v)
        pltpu.sync_copy(idx_hbm.at[pl.ds(base, ROWS_PER_WORKER)], idx_v)

        def issue_gathers(chunk_i, b):
            for off, width in GATHER_SPLITS:
                pltpu.async_copy(
                    table_hbm.at[idx_v.at[pl.ds(chunk_i * CHUNK + off,
                                                width)]],
                    bufs[b].at[pl.ds(off, width)],
                    gsems[b],
                )

        def finish_chunk(chunk_i, b):
            for off, width in GATHER_SPLITS:
                pltpu.make_async_copy(
                    table_hbm.at[idx_v.at[pl.ds(off, width)]],
                    bufs[b].at[pl.ds(off, width)],
                    gsems[b],
                ).wait()

            def pe_body(r):
                for c in range(HDIM // 16):
                    plsc.addupdate(bufs[b].at[r, pl.ds(c * 16, 16)],
                                   pe_v[r, pl.ds(c * 16, 16)])
            pl.loop(0, CHUNK, unroll=4)(pe_body)

            pltpu.sync_copy(bufs[b],
                            out_hbm.at[pl.ds(base + chunk_i * CHUNK, CHUNK)])

        issue_gathers(0, 0)

        def body(ii):
            for b in range(2):
                chunk_i = ii + b
                issue_gathers(chunk_i + 1, 1 - b)
                finish_chunk(chunk_i, b)
        pl.loop(0, NCHUNK - 2, step=2)(body)

        issue_gathers(NCHUNK - 1, 1)
        finish_chunk(NCHUNK - 2, 0)
        finish_chunk(NCHUNK - 1, 1)

    return k


def kernel(text, table):
    idx = text.reshape(ROWS).astype(jnp.int32)
    pe = _sinusoidal_pe(SEQLEN, HDIM)
    out = _make_sc_kernel()(idx, pe, table)
    return out.reshape(BATCH, SEQLEN, HDIM)
